# Initial kernel scaffold; baseline (speedup 1.0000x reference)
import jax
import jax.numpy as jnp
from jax.experimental import pallas as pl


def kernel(walk, neg_walk, emb):
    raise NotImplementedError("write your pallas kernel here")


# speedup vs baseline: 1.0659x; 1.0659x over previous
import functools

import jax
import jax.numpy as jnp
from jax import lax
from jax.experimental import pallas as pl
from jax.experimental.pallas import tpu as pltpu
from jax.experimental.pallas import tpu_sc as plsc

L = 16


def _sc_body(nw, n_chunks, cb, k, d, comb_hbm, emb_hbm, es_hbm, pos_hbm,
             idx_v, rows_v, es_v, pos_v, gsem):
  nc = 2
  wid = lax.axis_index("s") * nc + lax.axis_index("c")
  rows_per_chunk = cb * k
  n_dmas = rows_per_chunk // 128
  base_elem = wid * (n_chunks * cb)

  pos_v[...] = jnp.zeros((L,), jnp.float32)
  iota = lax.iota(jnp.int32, L)

  @pl.loop(0, n_chunks)
  def _chunk(c):
    pltpu.sync_copy(
        comb_hbm.at[pl.ds((base_elem + c * cb) * k, rows_per_chunk)], idx_v)
    descs = []
    for j in range(n_dmas):
      descs.append(pltpu.async_copy(
          emb_hbm.at[idx_v.at[pl.ds(j * 128, 128)]],
          rows_v.at[pl.ds(j * 128, 128)], gsem))
    for dsc in descs:
      dsc.wait()

    @pl.loop(0, cb)
    def _elem(b):
      r0 = b * k
      g0 = r0 + 1 + iota
      g1 = r0 + 17 + iota
      g2 = jnp.minimum(r0 + 33 + iota, rows_per_chunk - 1)
      acc0 = jnp.zeros((L,), jnp.float32)
      acc1 = jnp.zeros((L,), jnp.float32)
      acc2 = jnp.zeros((L,), jnp.float32)
      start_vecs = [rows_v[r0, pl.ds(i * L, L)] for i in range(d // L)]
      for dd in range(d):
        s = start_vecs[dd // L][dd % L]
        cold = jnp.full((L,), dd, jnp.int32)
        acc0 += s * plsc.load_gather(rows_v, [g0, cold])
        acc1 += s * plsc.load_gather(rows_v, [g1, cold])
        acc2 += s * plsc.load_gather(rows_v, [g2, cold])
      e = (jnp.exp(acc0) + jnp.exp(acc1)
           + jnp.where(iota < 7, jnp.exp(acc2), 0.0))
      pos_v[...] = pos_v[...] + acc0 + jnp.where(iota < 3, acc1, 0.0)
      es_v[...] = jnp.where(iota == b, jnp.sum(e), es_v[...])

    pltpu.sync_copy(es_v, es_hbm.at[pl.ds(base_elem + c * cb, cb)])

  pltpu.sync_copy(pos_v, pos_hbm.at[wid])


def _tc_body(mult, es_ref, pos_ref, out_ref):
  total = mult * jnp.sum(jnp.log(es_ref[...])) - jnp.sum(pos_ref[...])
  out_ref[...] = jnp.full((1, 1), 0.0, jnp.float32) + total


def kernel(walk, neg_walk, emb):
  b, w = walk.shape
  n = neg_walk.shape[1]
  d = emb.shape[1]
  k = w + n
  mesh = plsc.VectorSubcoreMesh(core_axis_name="c", subcore_axis_name="s")
  nw = mesh.num_cores * mesh.num_subcores
  cb = 16
  n_chunks = b // (nw * cb)
  rows_per_chunk = cb * k

  comb = jnp.concatenate([walk, neg_walk], axis=1).reshape(b * k)

  sc = pl.kernel(
      functools.partial(_sc_body, nw, n_chunks, cb, k, d),
      out_type=[
          jax.ShapeDtypeStruct((b,), jnp.float32),
          jax.ShapeDtypeStruct((nw, L), jnp.float32),
      ],
      mesh=mesh,
      compiler_params=pltpu.CompilerParams(needs_layout_passes=False,
                                           use_tc_tiling_on_sc=False),
      scratch_types=[
          pltpu.VMEM((rows_per_chunk,), jnp.int32),
          pltpu.VMEM((rows_per_chunk, d), jnp.float32),
          pltpu.VMEM((L,), jnp.float32),
          pltpu.VMEM((L,), jnp.float32),
          pltpu.SemaphoreType.DMA,
      ],
  )
  es, pos = sc(comb, emb)

  out = pl.pallas_call(
      functools.partial(_tc_body, float(w)),
      out_shape=jax.ShapeDtypeStruct((1, 1), jnp.float32),
  )(es.reshape(128, b // 128), pos.reshape(nw * L // 128, 128))
  return out[0, 0]
